# baseline (speedup 1.0000x reference)
import functools

import jax
import jax.numpy as jnp
from jax.experimental import pallas as pl


def _copy_body(wpe_ref, o_ref):
    o_ref[0] = wpe_ref[...]


@functools.cache
def _tc_copy(seq_len: int, emb_dim: int):
    n_blk = 5
    blk = seq_len // n_blk
    assert blk % 8 == 0 and blk * n_blk == seq_len
    return pl.pallas_call(
        _copy_body,
        grid=(n_blk,),
        in_specs=[pl.BlockSpec((blk, emb_dim), lambda i: (i, 0))],
        out_specs=pl.BlockSpec((1, blk, emb_dim), lambda i: (0, i, 0)),
        out_shape=jax.ShapeDtypeStruct((1, seq_len, emb_dim), jnp.float32),
    )


def kernel(x, wpe):
    seq_len = x.shape[1]
    emb_dim = wpe.shape[1]
    return _tc_copy(seq_len, emb_dim)(wpe[:seq_len])


# speedup vs baseline: 1.9414x; 1.9414x over previous
import functools

import jax
import jax.numpy as jnp
from jax.experimental import pallas as pl
from jax.experimental.pallas import tpu as pltpu


def _copy_body(w_ref, o_ref, buf, s0, s1, s2, s3):
    rows = w_ref.shape[0]
    half = (rows // 2 + 7) // 8 * 8
    rest = rows - half
    in0 = pltpu.make_async_copy(
        w_ref.at[pl.ds(0, half)], buf.at[pl.ds(0, half)], s0
    )
    in1 = pltpu.make_async_copy(
        w_ref.at[pl.ds(half, rest)], buf.at[pl.ds(half, rest)], s1
    )
    in0.start()
    in1.start()
    in0.wait()
    out0 = pltpu.make_async_copy(
        buf.at[pl.ds(0, half)], o_ref.at[0, pl.ds(0, half)], s2
    )
    out0.start()
    in1.wait()
    out1 = pltpu.make_async_copy(
        buf.at[pl.ds(half, rest)], o_ref.at[0, pl.ds(half, rest)], s3
    )
    out1.start()
    out0.wait()
    out1.wait()


@functools.cache
def _tc_copy(seq_len: int, emb_dim: int):
    return pl.pallas_call(
        _copy_body,
        out_shape=jax.ShapeDtypeStruct((1, seq_len, emb_dim), jnp.float32),
        in_specs=[pl.BlockSpec(memory_space=pl.ANY)],
        out_specs=pl.BlockSpec(memory_space=pl.ANY),
        scratch_shapes=[
            pltpu.VMEM((seq_len, emb_dim), jnp.float32),
            pltpu.SemaphoreType.DMA,
            pltpu.SemaphoreType.DMA,
            pltpu.SemaphoreType.DMA,
            pltpu.SemaphoreType.DMA,
        ],
    )


def kernel(x, wpe):
    seq_len = x.shape[1]
    emb_dim = wpe.shape[1]
    return _tc_copy(seq_len, emb_dim)(wpe[:seq_len])


# speedup vs baseline: 1.9583x; 1.0087x over previous
import functools

import jax
import jax.numpy as jnp
from jax.experimental import pallas as pl
from jax.experimental.pallas import tpu as pltpu


def _copy_body(w_ref, o_ref, sem):
    copy = pltpu.make_async_copy(w_ref, o_ref.at[0], sem)
    copy.start()
    copy.wait()


@functools.cache
def _tc_copy(seq_len: int, emb_dim: int):
    return pl.pallas_call(
        _copy_body,
        out_shape=jax.ShapeDtypeStruct((1, seq_len, emb_dim), jnp.float32),
        out_specs=pl.BlockSpec(memory_space=pl.ANY),
        scratch_shapes=[pltpu.SemaphoreType.DMA],
    )


def kernel(x, wpe):
    seq_len = x.shape[1]
    emb_dim = wpe.shape[1]
    return _tc_copy(seq_len, emb_dim)(wpe[:seq_len])


# speedup vs baseline: 1.9679x; 1.0049x over previous
import functools

import jax
import jax.numpy as jnp
from jax.experimental import pallas as pl
from jax.experimental.pallas import tpu as pltpu


def _copy_body(w_ref, o_ref, sem):
    copy = pltpu.make_async_copy(w_ref, o_ref.at[0], sem)
    copy.start()
    copy.wait()


@functools.cache
def _tc_copy(seq_len: int, emb_dim: int):
    return pl.pallas_call(
        _copy_body,
        out_shape=jax.ShapeDtypeStruct((1, seq_len, emb_dim), jnp.float32),
        out_specs=pl.BlockSpec(memory_space=pl.ANY),
        scratch_shapes=[pltpu.SemaphoreType.DMA],
        compiler_params=pltpu.CompilerParams(
            disable_bounds_checks=True,
            disable_semaphore_checks=True,
            skip_device_barrier=True,
        ),
    )


def kernel(x, wpe):
    seq_len = x.shape[1]
    emb_dim = wpe.shape[1]
    return _tc_copy(seq_len, emb_dim)(wpe[:seq_len])
